# baseline (speedup 1.0000x reference)
import jax
import jax.numpy as jnp
from jax.experimental import pallas as pl
from jax.experimental.pallas import tpu as pltpu
from jax.experimental.pallas import tpu_sc as plsc

_B, _EMB = 1024, 768
_M, _K, _D = 96, 256, 8
_BB = 128
_D2 = 2 * _D
_CD = 6 * _D2
_NIDX = _M * _B
_GW = 128


def _trunc16(x):
    return jax.lax.bitcast_convert_type(
        jax.lax.bitcast_convert_type(x, jnp.uint32) & jnp.uint32(0xFFFF0000),
        jnp.float32)


def _limbs(x):
    h = _trunc16(x)
    r = x - h
    m = _trunc16(r)
    l = r - m
    return (h.astype(jnp.bfloat16), m.astype(jnp.bfloat16),
            l.astype(jnp.bfloat16))


def _tc_body(vx_ref, lhs_ref, idx_ref):
    vx = vx_ref[...]
    lhs = lhs_ref[...]
    vh, vm, vl = _limbs(vx)
    rhs = jnp.concatenate([vh, vm, vh, vl, vh, vm], axis=1)
    adj = jax.lax.dot_general(
        lhs, rhs, (((2,), (1,)), ((0,), (0,))),
        preferred_element_type=jnp.float32,
        precision=jax.lax.Precision.DEFAULT)
    amax = jnp.max(adj, axis=1, keepdims=True)
    iota_k = jax.lax.broadcasted_iota(jnp.int32, adj.shape, 1)
    idx = jnp.min(jnp.where(adj == amax, iota_k, _K), axis=1)
    base = jax.lax.broadcasted_iota(jnp.int32, idx.shape, 0) * _K
    idx_ref[...] = idx + base


def _tc_indices(vx, lhs):
    return pl.pallas_call(
        _tc_body,
        grid=(_B // _BB,),
        in_specs=[
            pl.BlockSpec((_M, _D2, _BB), lambda i: (0, 0, i)),
            pl.BlockSpec((_M, _K, _CD), lambda i: (0, 0, 0)),
        ],
        out_specs=pl.BlockSpec((_M, _BB), lambda i: (0, i)),
        out_shape=jax.ShapeDtypeStruct((_M, _B), jnp.int32),
    )(vx, lhs)


def _sc_gather(cb_pad, indices):
    mesh = plsc.VectorSubcoreMesh(core_axis_name="c", subcore_axis_name="s")

    @pl.kernel(out_type=jax.ShapeDtypeStruct((_NIDX, 128), jnp.float32),
               mesh=mesh)
    def gather_kernel(cb_hbm, i_hbm, o_hbm):
        def body(i_vmem, o_vmem):
            pltpu.sync_copy(cb_hbm.at[i_vmem.at[0]], o_vmem)

        pltpu.emit_pipeline(
            body,
            grid=(_NIDX // _GW,),
            in_specs=[pl.BlockSpec((1, _GW), lambda i: (0, i))],
            out_specs=[pl.BlockSpec((_GW, 128), lambda i: (i, 0))],
            core_axis_name=("c", "s"),
            dimension_semantics=(pltpu.PARALLEL,),
        )(i_hbm, o_hbm)

    return gather_kernel(cb_pad, indices)


def kernel(vecs, codebook):
    vt = vecs.reshape(_B, _M, _D).transpose(1, 2, 0)
    ones = jnp.ones((_M, _D, _B), dtype=jnp.float32)
    vx = jnp.concatenate([vt, ones], axis=1)
    cbx = jnp.concatenate([codebook, -0.5 * codebook * codebook],
                          axis=2)
    ch, cm, cl = _limbs(cbx)
    lhs = jnp.concatenate([ch, ch, cm, ch, cl, cm], axis=2)
    idx = _tc_indices(vx, lhs)
    cb_pad = jnp.pad(codebook.reshape(_M * _K, _D),
                     ((0, 0), (0, 128 - _D)))
    q = _sc_gather(cb_pad, idx.reshape(1, _NIDX))
    q = q[:, :_D]
    return q.reshape(_M, _B, _D).transpose(1, 0, 2).reshape(_B, _EMB)


# speedup vs baseline: 1.0239x; 1.0239x over previous
import jax
import jax.numpy as jnp
from jax.experimental import pallas as pl
from jax.experimental.pallas import tpu as pltpu
from jax.experimental.pallas import tpu_sc as plsc

_B, _EMB = 1024, 768
_M, _K, _D = 96, 256, 8
_BB = 128
_D2 = 2 * _D
_CD = 6 * _D2
_NIDX = _M * _B
_GW = 256


def _trunc16(x):
    return jax.lax.bitcast_convert_type(
        jax.lax.bitcast_convert_type(x, jnp.uint32) & jnp.uint32(0xFFFF0000),
        jnp.float32)


def _limbs(x):
    h = _trunc16(x)
    r = x - h
    m = _trunc16(r)
    l = r - m
    return (h.astype(jnp.bfloat16), m.astype(jnp.bfloat16),
            l.astype(jnp.bfloat16))


def _tc_body(vx_ref, lhs_ref, idx_ref):
    vx = vx_ref[...]
    lhs = lhs_ref[...]
    vh, vm, vl = _limbs(vx)
    rhs = jnp.concatenate([vh, vm, vh, vl, vh, vm], axis=1)
    adj = jax.lax.dot_general(
        lhs, rhs, (((2,), (1,)), ((0,), (0,))),
        preferred_element_type=jnp.float32,
        precision=jax.lax.Precision.DEFAULT)
    amax = jnp.max(adj, axis=1, keepdims=True)
    iota_k = jax.lax.broadcasted_iota(jnp.int32, adj.shape, 1)
    idx = jnp.min(jnp.where(adj == amax, iota_k, _K), axis=1)
    base = jax.lax.broadcasted_iota(jnp.int32, idx.shape, 0) * _K
    idx_ref[...] = idx + base


def _tc_indices(vx, lhs):
    return pl.pallas_call(
        _tc_body,
        grid=(_B // _BB,),
        in_specs=[
            pl.BlockSpec((_M, _D2, _BB), lambda i: (0, 0, i)),
            pl.BlockSpec((_M, _K, _CD), lambda i: (0, 0, 0)),
        ],
        out_specs=pl.BlockSpec((_M, _BB), lambda i: (0, i)),
        out_shape=jax.ShapeDtypeStruct((_M, _B), jnp.int32),
    )(vx, lhs)


def _sc_gather(cb_pad, indices):
    mesh = plsc.VectorSubcoreMesh(core_axis_name="c", subcore_axis_name="s")

    @pl.kernel(out_type=jax.ShapeDtypeStruct((_NIDX, 128), jnp.float32),
               mesh=mesh)
    def gather_kernel(cb_hbm, i_hbm, o_hbm):
        def body(i_vmem, o_vmem):
            pltpu.sync_copy(cb_hbm.at[i_vmem.at[0]], o_vmem)

        pltpu.emit_pipeline(
            body,
            grid=(_NIDX // _GW,),
            in_specs=[pl.BlockSpec((1, _GW), lambda i: (0, i))],
            out_specs=[pl.BlockSpec((_GW, 128), lambda i: (i, 0))],
            core_axis_name=("c", "s"),
            dimension_semantics=(pltpu.PARALLEL,),
        )(i_hbm, o_hbm)

    return gather_kernel(cb_pad, indices)


def kernel(vecs, codebook):
    vt = vecs.reshape(_B, _M, _D).transpose(1, 2, 0)
    ones = jnp.ones((_M, _D, _B), dtype=jnp.float32)
    vx = jnp.concatenate([vt, ones], axis=1)
    cbx = jnp.concatenate([codebook, -0.5 * codebook * codebook],
                          axis=2)
    ch, cm, cl = _limbs(cbx)
    lhs = jnp.concatenate([ch, ch, cm, ch, cl, cm], axis=2)
    idx = _tc_indices(vx, lhs)
    cb_pad = jnp.pad(codebook.reshape(_M * _K, _D),
                     ((0, 0), (0, 128 - _D)))
    q = _sc_gather(cb_pad, idx.reshape(1, _NIDX))
    q = q[:, :_D]
    return q.reshape(_M, _B, _D).transpose(1, 0, 2).reshape(_B, _EMB)


# speedup vs baseline: 2.1313x; 2.0815x over previous
import jax
import jax.numpy as jnp
from jax.experimental import pallas as pl

_B, _EMB = 1024, 768
_M, _K, _D = 96, 256, 8
_BB = 256


def _body(vt_ref, cbx_ref, cbt_ref, out_ref):
    v = vt_ref[...]
    cbx = cbx_ref[...]
    cbt = cbt_ref[...]
    ones = jnp.ones((_M, _D, _BB), dtype=jnp.float32)
    vx = jnp.concatenate([v, ones], axis=1)
    adj = jax.lax.dot_general(
        cbx, vx, (((2,), (1,)), ((0,), (0,))),
        preferred_element_type=jnp.float32,
        precision=jax.lax.Precision.HIGHEST)
    amax = jnp.max(adj, axis=1, keepdims=True)
    oh = (adj == amax).astype(jnp.bfloat16)
    out_ref[...] = jax.lax.dot_general(
        cbt, oh, (((2,), (1,)), ((0,), (0,))),
        preferred_element_type=jnp.float32,
        precision=jax.lax.Precision.DEFAULT)


def kernel(vecs, codebook):
    vt = vecs.reshape(_B, _M, _D).transpose(1, 2, 0)
    cbx = jnp.concatenate([codebook, -0.5 * codebook * codebook],
                          axis=2)
    cbt = codebook.transpose(0, 2, 1).astype(jnp.bfloat16)
    q = pl.pallas_call(
        _body,
        grid=(_B // _BB,),
        in_specs=[
            pl.BlockSpec((_M, _D, _BB), lambda i: (0, 0, i)),
            pl.BlockSpec((_M, _K, 2 * _D), lambda i: (0, 0, 0)),
            pl.BlockSpec((_M, _D, _K), lambda i: (0, 0, 0)),
        ],
        out_specs=pl.BlockSpec((_M, _D, _BB), lambda i: (0, 0, i)),
        out_shape=jax.ShapeDtypeStruct((_M, _D, _B), jnp.float32),
    )(vt, cbx, cbt)
    return q.transpose(2, 0, 1).reshape(_B, _EMB)
